# baseline (speedup 1.0000x reference)
import jax
import jax.numpy as jnp
from jax.experimental import pallas as pl
from jax.experimental.pallas import tpu as pltpu
from jax.experimental.pallas import tpu_sc as plsc

D_MODEL = 64
WINDOW = 128


def _make_gather(num_indices: int):
    mesh = plsc.VectorSubcoreMesh(core_axis_name="core", subcore_axis_name="subcore")

    @jax.jit
    def gather(table, indices):
        indices = indices.reshape((1, num_indices))

        @pl.kernel(
            out_type=jax.ShapeDtypeStruct((num_indices, D_MODEL), table.dtype),
            mesh=mesh,
            compiler_params=pltpu.CompilerParams(use_tc_tiling_on_sc=False),
        )
        def k(table_hbm, idx_hbm, out_hbm):
            def body(idx_vmem, out_vmem):
                pltpu.sync_copy(table_hbm.at[idx_vmem.at[0]], out_vmem)

            pltpu.emit_pipeline(
                body,
                grid=(num_indices // WINDOW,),
                in_specs=[
                    pl.BlockSpec((1, WINDOW), index_map=lambda i: (0, i)),
                ],
                out_specs=[
                    pl.BlockSpec((WINDOW, D_MODEL), index_map=lambda i: (i, 0)),
                ],
                core_axis_name=("core", "subcore"),
                dimension_semantics=(pltpu.PARALLEL,),
            )(idx_hbm, out_hbm)

        return k(table, indices)

    return gather


def kernel(x, table):
    b0, b1 = x.shape
    flat = x.reshape(-1).astype(jnp.int32)
    out = _make_gather(b0 * b1)(table, flat)
    return out.reshape(b0, b1, D_MODEL)


# speedup vs baseline: 1.0466x; 1.0466x over previous
import jax
import jax.numpy as jnp
from jax.experimental import pallas as pl
from jax.experimental.pallas import tpu as pltpu
from jax.experimental.pallas import tpu_sc as plsc

D_MODEL = 64
WINDOW = 256


def _make_gather(num_indices: int):
    mesh = plsc.VectorSubcoreMesh(core_axis_name="core", subcore_axis_name="subcore")

    @jax.jit
    def gather(table, indices):
        indices = indices.reshape((1, num_indices))

        @pl.kernel(
            out_type=jax.ShapeDtypeStruct((num_indices, D_MODEL), table.dtype),
            mesh=mesh,
            compiler_params=pltpu.CompilerParams(use_tc_tiling_on_sc=False),
        )
        def k(table_hbm, idx_hbm, out_hbm):
            def body(idx_vmem, out_vmem):
                pltpu.sync_copy(table_hbm.at[idx_vmem.at[0]], out_vmem)

            pltpu.emit_pipeline(
                body,
                grid=(num_indices // WINDOW,),
                in_specs=[
                    pl.BlockSpec((1, WINDOW), index_map=lambda i: (0, i)),
                ],
                out_specs=[
                    pl.BlockSpec((WINDOW, D_MODEL), index_map=lambda i: (i, 0)),
                ],
                core_axis_name=("core", "subcore"),
                dimension_semantics=(pltpu.PARALLEL,),
            )(idx_hbm, out_hbm)

        return k(table, indices)

    return gather


def kernel(x, table):
    b0, b1 = x.shape
    flat = x.reshape(-1).astype(jnp.int32)
    out = _make_gather(b0 * b1)(table, flat)
    return out.reshape(b0, b1, D_MODEL)
